# baseline (speedup 1.0000x reference)
import functools

import jax
import jax.numpy as jnp
from jax import lax
from jax.experimental import pallas as pl
from jax.experimental.pallas import tpu as pltpu
from jax.experimental.pallas import tpu_sc as plsc

N = 10000
E = 320000
IN = 128
HID = 128
CLS = 16
R = 8
C = 4

NC = 2
NS = 16
LANES = 16
NW = NC * NS

K_EDGE = 80
NBUF = 4
E_TOT = E + N
NCH = -(-E_TOT // (NW * K_EDGE * NBUF)) * NBUF
EDGES_PER_TILE = NCH * K_EDGE
E_PAD = EDGES_PER_TILE * NW
N_PAD = 10240
N_PER_TILE = N_PAD // NS


def _proj_body(ck, d, pad, comp_ref, bases_ref, h_ref, out_ref):
    r = pl.program_id(0)
    w = comp_ref[r, 0] * bases_ref[0]
    for c in range(1, ck):
        w = w + comp_ref[r, c] * bases_ref[c]
    m = jnp.dot(h_ref[...], w, preferred_element_type=jnp.float32)
    if pad:
        m = jnp.concatenate(
            [m, jnp.zeros((m.shape[0], pad), jnp.float32)], axis=1)
    out_ref[0] = m


def _proj(comp_ext, bases_ext, h, bn=1000):
    rk, ck = comp_ext.shape
    d = bases_ext.shape[-1]
    nb = N // bn
    return pl.pallas_call(
        functools.partial(_proj_body, ck, d, HID - d),
        grid=(rk, nb),
        in_specs=[
            pl.BlockSpec(memory_space=pltpu.SMEM),
            pl.BlockSpec((ck, IN, d), lambda r, b: (0, 0, 0)),
            pl.BlockSpec((bn, IN), lambda r, b: (b, 0)),
        ],
        out_specs=pl.BlockSpec((1, bn, HID), lambda r, b: (r, b, 0)),
        out_shape=jax.ShapeDtypeStruct((rk, N, HID), jnp.float32),
    )(comp_ext, bases_ext, h)


def _combine_body(relu, dout, p_ref, b_ref, out_ref):
    acc = p_ref[0] + p_ref[1]
    acc = acc[:, :dout] + b_ref[...]
    out_ref[...] = jnp.maximum(acc, 0.0) if relu else acc


def _combine(partials, bias_row, relu, bn, nrows, dout):
    nb = nrows // bn
    d = partials.shape[-1]
    return pl.pallas_call(
        functools.partial(_combine_body, relu, dout),
        grid=(nb,),
        in_specs=[
            pl.BlockSpec((2, bn, d), lambda b: (0, b, 0)),
            pl.BlockSpec((1, dout), lambda b: (0, 0)),
        ],
        out_specs=pl.BlockSpec((bn, dout), lambda b: (b, 0)),
        out_shape=jax.ShapeDtypeStruct((nrows, dout), jnp.float32),
    )(partials, bias_row)


@functools.lru_cache(maxsize=None)
def _make_sc_scatter(d):
    mesh = plsc.VectorSubcoreMesh(
        core_axis_name="c", subcore_axis_name="s", num_cores=NC,
        num_subcores=NS)

    @functools.partial(
        pl.kernel,
        out_type=jax.ShapeDtypeStruct((NC, N_PAD, d), jnp.float32),
        mesh=mesh,
        scratch_types=(
            [pltpu.VMEM((K_EDGE,), jnp.int32) for _ in range(NBUF)]
            + [pltpu.VMEM((K_EDGE,), jnp.int32) for _ in range(NBUF)]
            + [pltpu.VMEM((K_EDGE,), jnp.float32) for _ in range(NBUF)]
            + [
                pltpu.VMEM((NBUF, K_EDGE, d), jnp.float32),
                pltpu.VMEM_SHARED((N_PAD, d), jnp.float32),
                pltpu.SemaphoreType.DMA((NBUF,)),
                pltpu.SemaphoreType.DMA((NBUF,)),
                pltpu.SemaphoreType.DMA((NBUF,)),
            ]
        ),
    )
    def sc_scatter(table, gidx, dst, norm, zeros, out, *sc):
        gbufs, dbufs, nbufs = sc[:NBUF], sc[NBUF:2 * NBUF], sc[2 * NBUF:3 * NBUF]
        msg, agg_sh, isem, gsem, ssem = sc[3 * NBUF:]
        cid = lax.axis_index("c")
        sid = lax.axis_index("s")
        wid = cid * NS + sid
        base0 = wid * EDGES_PER_TILE

        def idx_copies(i, s):
            off = base0 + i * K_EDGE
            return (
                pltpu.make_async_copy(
                    gidx.at[pl.ds(off, K_EDGE)], gbufs[s], isem.at[s]),
                pltpu.make_async_copy(
                    dst.at[pl.ds(off, K_EDGE)], dbufs[s], isem.at[s]),
                pltpu.make_async_copy(
                    norm.at[pl.ds(off, K_EDGE)], nbufs[s], isem.at[s]),
            )

        def idx_load(i, s):
            for cp in idx_copies(i, s):
                cp.start()

        def idx_wait(s):
            for cp in idx_copies(0, s):
                cp.wait()

        def gather(s):
            return pltpu.make_async_copy(
                table.at[gbufs[s]], msg.at[s], gsem.at[s])

        def scatter(s):
            return pltpu.make_async_copy(
                msg.at[s], agg_sh.at[dbufs[s]], ssem.at[s])

        pltpu.sync_copy(zeros.at[pl.ds(sid * N_PER_TILE, N_PER_TILE)],
                        agg_sh.at[pl.ds(sid * N_PER_TILE, N_PER_TILE)])
        plsc.subcore_barrier()

        idx_load(0, 0)
        idx_load(1, 1)
        idx_load(2, 2)
        idx_wait(0)
        gather(0).start()
        idx_wait(1)
        gather(1).start()

        def group(g, carry):
            for b in range(NBUF):
                i = g * NBUF + b
                s_m1 = (b + NBUF - 1) % NBUF
                s_p2 = (b + 2) % NBUF
                gather(b).wait()

                def scale(q, c2):
                    nv = nbufs[b][pl.ds(q * LANES, LANES)]
                    for t in range(LANES):
                        nj = nv[t]
                        j = q * LANES + t
                        for w in range(d // LANES):
                            sl = pl.ds(w * LANES, LANES)
                            msg[b, j, sl] = msg[b, j, sl] * nj
                    return c2

                lax.fori_loop(0, K_EDGE // LANES, scale, 0)
                scatter(b).start(add=True)

                @pl.when(i > 0)
                def _():
                    scatter(s_m1).wait()

                @pl.when(i + 3 < NCH)
                def _():
                    idx_load(i + 3, s_m1)

                @pl.when(i + 2 < NCH)
                def _():
                    idx_wait(s_p2)
                    gather(s_p2).start()

            return carry

        lax.fori_loop(0, NCH // NBUF, group, 0)
        scatter(NBUF - 1).wait()
        plsc.subcore_barrier()
        pltpu.sync_copy(agg_sh.at[pl.ds(sid * N_PER_TILE, N_PER_TILE)],
                        out.at[cid, pl.ds(sid * N_PER_TILE, N_PER_TILE)])

    return sc_scatter


def kernel(x, edge_index, edge_types, norm, bases1, comp1, loop_w1, bias1,
           bases2, comp2, loop_w2, bias2):
    src = edge_index[0]
    dst = edge_index[1]
    ar = jnp.arange(N, dtype=jnp.int32)
    pad = E_PAD - E_TOT

    dst_all = jnp.concatenate(
        [dst, ar, jnp.zeros((pad,), jnp.int32)])
    norm_all = jnp.concatenate(
        [norm, jnp.ones((N,), jnp.float32), jnp.zeros((pad,), jnp.float32)])
    gidx = jnp.concatenate(
        [edge_types * N + src, R * N + ar, jnp.zeros((pad,), jnp.int32)])

    comp_ext1 = jnp.concatenate([
        jnp.concatenate([comp1, jnp.zeros((R, 1), jnp.float32)], axis=1),
        jnp.concatenate([jnp.zeros((1, C), jnp.float32),
                         jnp.ones((1, 1), jnp.float32)], axis=1),
    ], axis=0)
    comp_ext2 = jnp.concatenate([
        jnp.concatenate([comp2, jnp.zeros((R, 1), jnp.float32)], axis=1),
        jnp.concatenate([jnp.zeros((1, C), jnp.float32),
                         jnp.ones((1, 1), jnp.float32)], axis=1),
    ], axis=0)
    bases1_ext = jnp.concatenate([bases1, loop_w1[None]], axis=0)
    bases2_ext = jnp.concatenate([bases2, loop_w2[None]], axis=0)

    zeros128 = jnp.zeros((N_PAD, HID), jnp.float32)
    scat = _make_sc_scatter(HID)

    h1_tab = _proj(comp_ext1, bases1_ext, x)
    p1 = scat(h1_tab.reshape((R + 1) * N, HID), gidx, dst_all, norm_all,
              zeros128)
    h1 = _combine(p1, bias1.reshape(1, HID), relu=True, bn=1000, nrows=N,
                  dout=HID)

    h2_tab = _proj(comp_ext2, bases2_ext, h1)
    p2 = scat(h2_tab.reshape((R + 1) * N, HID), gidx, dst_all, norm_all,
              zeros128)
    return _combine(p2, bias2.reshape(1, CLS), relu=False, bn=1000, nrows=N,
                    dout=CLS)


# speedup vs baseline: 2.2745x; 2.2745x over previous
import functools

import jax
import jax.numpy as jnp
from jax import lax
from jax.experimental import pallas as pl
from jax.experimental.pallas import tpu as pltpu
from jax.experimental.pallas import tpu_sc as plsc

N = 10000
E = 320000
IN = 128
HID = 128
CLS = 16
R = 8
C = 4

NC = 2
NS = 16
LANES = 16
NW = NC * NS

K_EDGE = 80
NBUF = 4
E_TOT = E + N
NCH = -(-E_TOT // (NW * K_EDGE * NBUF)) * NBUF
EDGES_PER_TILE = NCH * K_EDGE
E_PAD = EDGES_PER_TILE * NW
N_PAD = 10240
N_PER_TILE = N_PAD // NS


def _proj_body(ck, d, pad, comp_ref, bases_ref, h_ref, out_ref):
    r = pl.program_id(0)
    w = comp_ref[r, 0] * bases_ref[0]
    for c in range(1, ck):
        w = w + comp_ref[r, c] * bases_ref[c]
    m = jnp.dot(h_ref[...], w, preferred_element_type=jnp.float32)
    if pad:
        m = jnp.concatenate(
            [m, jnp.zeros((m.shape[0], pad), jnp.float32)], axis=1)
    out_ref[0] = m


def _proj(comp_ext, bases_ext, h, bn=1000):
    rk, ck = comp_ext.shape
    d = bases_ext.shape[-1]
    nb = N // bn
    return pl.pallas_call(
        functools.partial(_proj_body, ck, d, HID - d),
        grid=(rk, nb),
        in_specs=[
            pl.BlockSpec(memory_space=pltpu.SMEM),
            pl.BlockSpec((ck, IN, d), lambda r, b: (0, 0, 0)),
            pl.BlockSpec((bn, IN), lambda r, b: (b, 0)),
        ],
        out_specs=pl.BlockSpec((1, bn, HID), lambda r, b: (r, b, 0)),
        out_shape=jax.ShapeDtypeStruct((rk, N, HID), jnp.float32),
    )(comp_ext, bases_ext, h)


def _combine_body(relu, dout, p_ref, b_ref, out_ref):
    acc = p_ref[0] + p_ref[1]
    acc = acc[:, :dout] + b_ref[...]
    out_ref[...] = jnp.maximum(acc, 0.0) if relu else acc


def _combine(partials, bias_row, relu, bn, nrows, dout):
    nb = nrows // bn
    d = partials.shape[-1]
    return pl.pallas_call(
        functools.partial(_combine_body, relu, dout),
        grid=(nb,),
        in_specs=[
            pl.BlockSpec((2, bn, d), lambda b: (0, b, 0)),
            pl.BlockSpec((1, dout), lambda b: (0, 0)),
        ],
        out_specs=pl.BlockSpec((bn, dout), lambda b: (b, 0)),
        out_shape=jax.ShapeDtypeStruct((nrows, dout), jnp.float32),
    )(partials, bias_row)


@functools.lru_cache(maxsize=None)
def _make_sc_scatter(d):
    mesh = plsc.VectorSubcoreMesh(
        core_axis_name="c", subcore_axis_name="s", num_cores=NC,
        num_subcores=NS)

    @functools.partial(
        pl.kernel,
        out_type=jax.ShapeDtypeStruct((NC, N_PAD, d), jnp.float32),
        mesh=mesh,
        scratch_types=(
            [pltpu.VMEM((K_EDGE,), jnp.int32) for _ in range(NBUF)]
            + [pltpu.VMEM((K_EDGE,), jnp.int32) for _ in range(NBUF)]
            + [pltpu.VMEM((K_EDGE,), jnp.float32) for _ in range(NBUF)]
            + [
                pltpu.VMEM((NBUF, K_EDGE, d), jnp.float32),
                pltpu.VMEM_SHARED((N_PAD, d), jnp.float32),
                pltpu.SemaphoreType.DMA((NBUF,)),
                pltpu.SemaphoreType.DMA((NBUF,)),
                pltpu.SemaphoreType.DMA((NBUF,)),
            ]
        ),
    )
    def sc_scatter(table, gidx, dst, norm, zeros, out, *sc):
        gbufs, dbufs, nbufs = sc[:NBUF], sc[NBUF:2 * NBUF], sc[2 * NBUF:3 * NBUF]
        msg, agg_sh, isem, gsem, ssem = sc[3 * NBUF:]
        cid = lax.axis_index("c")
        sid = lax.axis_index("s")
        wid = cid * NS + sid
        base0 = wid * EDGES_PER_TILE

        def idx_copies(i, s):
            off = base0 + i * K_EDGE
            return (
                pltpu.make_async_copy(
                    gidx.at[pl.ds(off, K_EDGE)], gbufs[s], isem.at[s]),
                pltpu.make_async_copy(
                    dst.at[pl.ds(off, K_EDGE)], dbufs[s], isem.at[s]),
                pltpu.make_async_copy(
                    norm.at[pl.ds(off, K_EDGE)], nbufs[s], isem.at[s]),
            )

        def idx_load(i, s):
            for cp in idx_copies(i, s):
                cp.start()

        def idx_wait(s):
            for cp in idx_copies(0, s):
                cp.wait()

        def gather(s):
            return pltpu.make_async_copy(
                table.at[gbufs[s]], msg.at[s], gsem.at[s])

        def scatter(s):
            return pltpu.make_async_copy(
                msg.at[s], agg_sh.at[dbufs[s]], ssem.at[s])

        pltpu.sync_copy(zeros.at[pl.ds(sid * N_PER_TILE, N_PER_TILE)],
                        agg_sh.at[pl.ds(sid * N_PER_TILE, N_PER_TILE)])
        plsc.subcore_barrier()

        idx_load(0, 0)
        idx_load(1, 1)
        idx_load(2, 2)
        idx_wait(0)
        gather(0).start()
        idx_wait(1)
        gather(1).start()

        def group(g, carry):
            for b in range(NBUF):
                i = g * NBUF + b
                s_m1 = (b + NBUF - 1) % NBUF
                s_p2 = (b + 2) % NBUF
                gather(b).wait()

                def scale(q, c2):
                    nv = nbufs[b][pl.ds(q * LANES, LANES)]
                    for t in range(LANES):
                        nj = nv[t]
                        j = q * LANES + t
                        for w in range(d // LANES):
                            sl = pl.ds(w * LANES, LANES)
                            msg[b, j, sl] = msg[b, j, sl] * nj
                    return c2

                lax.fori_loop(0, K_EDGE // LANES, scale, 0)
                scatter(b).start(add=True)

                @pl.when(i > 0)
                def _():
                    scatter(s_m1).wait()

                @pl.when(i + 3 < NCH)
                def _():
                    idx_load(i + 3, s_m1)

                @pl.when(i + 2 < NCH)
                def _():
                    idx_wait(s_p2)
                    gather(s_p2).start()

            return carry

        lax.fori_loop(0, NCH // NBUF, group, 0)
        scatter(NBUF - 1).wait()
        plsc.subcore_barrier()
        pltpu.sync_copy(agg_sh.at[pl.ds(sid * N_PER_TILE, N_PER_TILE)],
                        out.at[cid, pl.ds(sid * N_PER_TILE, N_PER_TILE)])

    return sc_scatter


def kernel(x, edge_index, edge_types, norm, bases1, comp1, loop_w1, bias1,
           bases2, comp2, loop_w2, bias2):
    src = edge_index[0]
    dst = edge_index[1]
    ar = jnp.arange(N, dtype=jnp.int32)
    pad = E_PAD - E_TOT

    arp = jnp.arange(pad, dtype=jnp.int32)
    dst_all = jnp.concatenate([dst, ar, arp % N_PAD])
    norm_all = jnp.concatenate(
        [norm, jnp.ones((N,), jnp.float32), jnp.zeros((pad,), jnp.float32)])
    gidx = jnp.concatenate([edge_types * N + src, R * N + ar, arp % N])

    comp_ext1 = jnp.concatenate([
        jnp.concatenate([comp1, jnp.zeros((R, 1), jnp.float32)], axis=1),
        jnp.concatenate([jnp.zeros((1, C), jnp.float32),
                         jnp.ones((1, 1), jnp.float32)], axis=1),
    ], axis=0)
    comp_ext2 = jnp.concatenate([
        jnp.concatenate([comp2, jnp.zeros((R, 1), jnp.float32)], axis=1),
        jnp.concatenate([jnp.zeros((1, C), jnp.float32),
                         jnp.ones((1, 1), jnp.float32)], axis=1),
    ], axis=0)
    bases1_ext = jnp.concatenate([bases1, loop_w1[None]], axis=0)
    bases2_ext = jnp.concatenate([bases2, loop_w2[None]], axis=0)

    zeros128 = jnp.zeros((N_PAD, HID), jnp.float32)
    scat = _make_sc_scatter(HID)

    h1_tab = _proj(comp_ext1, bases1_ext, x)
    p1 = scat(h1_tab.reshape((R + 1) * N, HID), gidx, dst_all, norm_all,
              zeros128)
    h1 = _combine(p1, bias1.reshape(1, HID), relu=True, bn=1000, nrows=N,
                  dout=HID)

    h2_tab = _proj(comp_ext2, bases2_ext, h1)
    p2 = scat(h2_tab.reshape((R + 1) * N, HID), gidx, dst_all, norm_all,
              zeros128)
    return _combine(p2, bias2.reshape(1, CLS), relu=False, bn=1000, nrows=N,
                    dout=CLS)


# speedup vs baseline: 2.4564x; 1.0800x over previous
import functools

import jax
import jax.numpy as jnp
from jax import lax
from jax.experimental import pallas as pl
from jax.experimental.pallas import tpu as pltpu
from jax.experimental.pallas import tpu_sc as plsc

N = 10000
E = 320000
IN = 128
HID = 128
CLS = 16
R = 8
C = 4

NC = 2
NS = 16
LANES = 16
NW = NC * NS

K_EDGE = 80
NBUF = 4
E_TOT = E + N
NCH = -(-E_TOT // (NW * K_EDGE * NBUF)) * NBUF
EDGES_PER_TILE = NCH * K_EDGE
E_PAD = EDGES_PER_TILE * NW
N_PAD = 10240
N_PER_TILE = N_PAD // NS


def _proj_body(ck, d, pad, comp_ref, bases_ref, h_ref, out_ref):
    r = pl.program_id(1)
    w = comp_ref[r, 0] * bases_ref[0]
    for c in range(1, ck):
        w = w + comp_ref[r, c] * bases_ref[c]
    m = jnp.dot(h_ref[...], w, preferred_element_type=jnp.float32)
    if pad:
        m = jnp.concatenate(
            [m, jnp.zeros((m.shape[0], pad), jnp.float32)], axis=1)
    out_ref[0] = m


def _proj(comp_ext, bases_ext, h, bn=1000):
    rk, ck = comp_ext.shape
    d = bases_ext.shape[-1]
    nb = N // bn
    return pl.pallas_call(
        functools.partial(_proj_body, ck, d, HID - d),
        grid=(nb, rk),
        in_specs=[
            pl.BlockSpec(memory_space=pltpu.SMEM),
            pl.BlockSpec((ck, IN, d), lambda b, r: (0, 0, 0)),
            pl.BlockSpec((bn, IN), lambda b, r: (b, 0)),
        ],
        out_specs=pl.BlockSpec((1, bn, HID), lambda b, r: (r, b, 0)),
        out_shape=jax.ShapeDtypeStruct((rk, N, HID), jnp.float32),
    )(comp_ext, bases_ext, h)


def _combine_body(relu, dout, p_ref, b_ref, out_ref):
    acc = p_ref[0] + p_ref[1]
    acc = acc[:, :dout] + b_ref[...]
    out_ref[...] = jnp.maximum(acc, 0.0) if relu else acc


def _combine(partials, bias_row, relu, bn, nrows, dout):
    nb = nrows // bn
    d = partials.shape[-1]
    return pl.pallas_call(
        functools.partial(_combine_body, relu, dout),
        grid=(nb,),
        in_specs=[
            pl.BlockSpec((2, bn, d), lambda b: (0, b, 0)),
            pl.BlockSpec((1, dout), lambda b: (0, 0)),
        ],
        out_specs=pl.BlockSpec((bn, dout), lambda b: (b, 0)),
        out_shape=jax.ShapeDtypeStruct((nrows, dout), jnp.float32),
    )(partials, bias_row)


@functools.lru_cache(maxsize=None)
def _make_sc_scatter(d):
    mesh = plsc.VectorSubcoreMesh(
        core_axis_name="c", subcore_axis_name="s", num_cores=NC,
        num_subcores=NS)

    @functools.partial(
        pl.kernel,
        out_type=jax.ShapeDtypeStruct((NC, N_PAD, d), jnp.float32),
        mesh=mesh,
        scratch_types=(
            [pltpu.VMEM((K_EDGE,), jnp.int32) for _ in range(NBUF)]
            + [pltpu.VMEM((K_EDGE,), jnp.int32) for _ in range(NBUF)]
            + [pltpu.VMEM((K_EDGE,), jnp.float32) for _ in range(NBUF)]
            + [
                pltpu.VMEM((NBUF, K_EDGE, d), jnp.float32),
                pltpu.VMEM_SHARED((N_PAD, d), jnp.float32),
                pltpu.SemaphoreType.DMA((NBUF,)),
                pltpu.SemaphoreType.DMA((NBUF,)),
                pltpu.SemaphoreType.DMA((NBUF,)),
            ]
        ),
    )
    def sc_scatter(table, gidx, dst, norm, zeros, out, *sc):
        gbufs, dbufs, nbufs = sc[:NBUF], sc[NBUF:2 * NBUF], sc[2 * NBUF:3 * NBUF]
        msg, agg_sh, isem, gsem, ssem = sc[3 * NBUF:]
        cid = lax.axis_index("c")
        sid = lax.axis_index("s")
        wid = cid * NS + sid
        base0 = wid * EDGES_PER_TILE

        def idx_copies(i, s):
            off = base0 + i * K_EDGE
            return (
                pltpu.make_async_copy(
                    gidx.at[pl.ds(off, K_EDGE)], gbufs[s], isem.at[s]),
                pltpu.make_async_copy(
                    dst.at[pl.ds(off, K_EDGE)], dbufs[s], isem.at[s]),
                pltpu.make_async_copy(
                    norm.at[pl.ds(off, K_EDGE)], nbufs[s], isem.at[s]),
            )

        def idx_load(i, s):
            for cp in idx_copies(i, s):
                cp.start()

        def idx_wait(s):
            for cp in idx_copies(0, s):
                cp.wait()

        def gather(s):
            return pltpu.make_async_copy(
                table.at[gbufs[s]], msg.at[s], gsem.at[s])

        def scatter(s):
            return pltpu.make_async_copy(
                msg.at[s], agg_sh.at[dbufs[s]], ssem.at[s])

        pltpu.sync_copy(zeros.at[pl.ds(sid * N_PER_TILE, N_PER_TILE)],
                        agg_sh.at[pl.ds(sid * N_PER_TILE, N_PER_TILE)])
        plsc.subcore_barrier()

        idx_load(0, 0)
        idx_load(1, 1)
        idx_load(2, 2)
        idx_wait(0)
        gather(0).start()
        idx_wait(1)
        gather(1).start()

        def group(g, carry):
            for b in range(NBUF):
                i = g * NBUF + b
                s_m1 = (b + NBUF - 1) % NBUF
                s_p2 = (b + 2) % NBUF
                gather(b).wait()

                def scale(q, c2):
                    nv = nbufs[b][pl.ds(q * LANES, LANES)]
                    for t in range(LANES):
                        nj = nv[t]
                        j = q * LANES + t
                        for w in range(d // LANES):
                            sl = pl.ds(w * LANES, LANES)
                            msg[b, j, sl] = msg[b, j, sl] * nj
                    return c2

                lax.fori_loop(0, K_EDGE // LANES, scale, 0)
                scatter(b).start(add=True)

                @pl.when(i > 0)
                def _():
                    scatter(s_m1).wait()

                @pl.when(i + 3 < NCH)
                def _():
                    idx_load(i + 3, s_m1)

                @pl.when(i + 2 < NCH)
                def _():
                    idx_wait(s_p2)
                    gather(s_p2).start()

            return carry

        lax.fori_loop(0, NCH // NBUF, group, 0)
        scatter(NBUF - 1).wait()
        plsc.subcore_barrier()
        pltpu.sync_copy(agg_sh.at[pl.ds(sid * N_PER_TILE, N_PER_TILE)],
                        out.at[cid, pl.ds(sid * N_PER_TILE, N_PER_TILE)])

    return sc_scatter


def kernel(x, edge_index, edge_types, norm, bases1, comp1, loop_w1, bias1,
           bases2, comp2, loop_w2, bias2):
    src = edge_index[0]
    dst = edge_index[1]
    ar = jnp.arange(N, dtype=jnp.int32)
    pad = E_PAD - E_TOT

    arp = jnp.arange(pad, dtype=jnp.int32)
    dst_all = jnp.concatenate([dst, ar, arp % N_PAD])
    norm_all = jnp.concatenate(
        [norm, jnp.ones((N,), jnp.float32), jnp.zeros((pad,), jnp.float32)])
    gidx = jnp.concatenate([edge_types * N + src, R * N + ar, arp % N])

    comp_ext1 = jnp.concatenate([
        jnp.concatenate([comp1, jnp.zeros((R, 1), jnp.float32)], axis=1),
        jnp.concatenate([jnp.zeros((1, C), jnp.float32),
                         jnp.ones((1, 1), jnp.float32)], axis=1),
    ], axis=0)
    comp_ext2 = jnp.concatenate([
        jnp.concatenate([comp2, jnp.zeros((R, 1), jnp.float32)], axis=1),
        jnp.concatenate([jnp.zeros((1, C), jnp.float32),
                         jnp.ones((1, 1), jnp.float32)], axis=1),
    ], axis=0)
    bases1_ext = jnp.concatenate([bases1, loop_w1[None]], axis=0)
    bases2_ext = jnp.concatenate([bases2, loop_w2[None]], axis=0)

    zeros128 = jnp.zeros((N_PAD, HID), jnp.float32)
    scat = _make_sc_scatter(HID)

    h1_tab = _proj(comp_ext1, bases1_ext, x)
    p1 = scat(h1_tab.reshape((R + 1) * N, HID), gidx, dst_all, norm_all,
              zeros128)
    h1 = _combine(p1, bias1.reshape(1, HID), relu=True, bn=1000, nrows=N,
                  dout=HID)

    h2_tab = _proj(comp_ext2, bases2_ext, h1)
    p2 = scat(h2_tab.reshape((R + 1) * N, HID), gidx, dst_all, norm_all,
              zeros128)
    return _combine(p2, bias2.reshape(1, CLS), relu=False, bn=1000, nrows=N,
                    dout=CLS)


# speedup vs baseline: 2.4564x; 1.0000x over previous
import functools

import jax
import jax.numpy as jnp
from jax import lax
from jax.experimental import pallas as pl
from jax.experimental.pallas import tpu as pltpu
from jax.experimental.pallas import tpu_sc as plsc

N = 10000
E = 320000
IN = 128
HID = 128
CLS = 16
R = 8
C = 4

NC = 2
NS = 16
LANES = 16
NW = NC * NS

K_EDGE = 80
NBUF = 4
E_TOT = E + N
NCH = -(-E_TOT // (NW * K_EDGE * NBUF)) * NBUF
EDGES_PER_TILE = NCH * K_EDGE
E_PAD = EDGES_PER_TILE * NW
N_PAD = 10240
N_PER_TILE = N_PAD // NS


def _proj_body(ck, d, pad, comp_ref, bases_ref, h_ref, out_ref):
    r = pl.program_id(1)
    w = comp_ref[r, 0] * bases_ref[0]
    for c in range(1, ck):
        w = w + comp_ref[r, c] * bases_ref[c]
    m = jnp.dot(h_ref[...].astype(jnp.bfloat16), w.astype(jnp.bfloat16),
                preferred_element_type=jnp.float32)
    if pad:
        m = jnp.concatenate(
            [m, jnp.zeros((m.shape[0], pad), jnp.float32)], axis=1)
    out_ref[0] = m


def _proj(comp_ext, bases_ext, h, bn=1000):
    rk, ck = comp_ext.shape
    d = bases_ext.shape[-1]
    nb = N // bn
    return pl.pallas_call(
        functools.partial(_proj_body, ck, d, HID - d),
        grid=(nb, rk),
        in_specs=[
            pl.BlockSpec(memory_space=pltpu.SMEM),
            pl.BlockSpec((ck, IN, d), lambda b, r: (0, 0, 0)),
            pl.BlockSpec((bn, IN), lambda b, r: (b, 0)),
        ],
        out_specs=pl.BlockSpec((1, bn, HID), lambda b, r: (r, b, 0)),
        out_shape=jax.ShapeDtypeStruct((rk, N, HID), jnp.float32),
    )(comp_ext, bases_ext, h)


def _combine_body(relu, dout, p_ref, b_ref, out_ref):
    acc = p_ref[0] + p_ref[1]
    acc = acc[:, :dout] + b_ref[...]
    out_ref[...] = jnp.maximum(acc, 0.0) if relu else acc


def _combine(partials, bias_row, relu, bn, nrows, dout):
    nb = nrows // bn
    d = partials.shape[-1]
    return pl.pallas_call(
        functools.partial(_combine_body, relu, dout),
        grid=(nb,),
        in_specs=[
            pl.BlockSpec((2, bn, d), lambda b: (0, b, 0)),
            pl.BlockSpec((1, dout), lambda b: (0, 0)),
        ],
        out_specs=pl.BlockSpec((bn, dout), lambda b: (b, 0)),
        out_shape=jax.ShapeDtypeStruct((nrows, dout), jnp.float32),
    )(partials, bias_row)


@functools.lru_cache(maxsize=None)
def _make_sc_scatter(d):
    mesh = plsc.VectorSubcoreMesh(
        core_axis_name="c", subcore_axis_name="s", num_cores=NC,
        num_subcores=NS)

    @functools.partial(
        pl.kernel,
        out_type=jax.ShapeDtypeStruct((NC, N_PAD, d), jnp.float32),
        mesh=mesh,
        scratch_types=(
            [pltpu.VMEM((K_EDGE,), jnp.int32) for _ in range(NBUF)]
            + [pltpu.VMEM((K_EDGE,), jnp.int32) for _ in range(NBUF)]
            + [pltpu.VMEM((K_EDGE,), jnp.float32) for _ in range(NBUF)]
            + [
                pltpu.VMEM((NBUF, K_EDGE, d), jnp.float32),
                pltpu.VMEM_SHARED((N_PAD, d), jnp.float32),
                pltpu.SemaphoreType.DMA((NBUF,)),
                pltpu.SemaphoreType.DMA((NBUF,)),
                pltpu.SemaphoreType.DMA((NBUF,)),
            ]
        ),
    )
    def sc_scatter(table, gidx, dst, norm, zeros, out, *sc):
        gbufs, dbufs, nbufs = sc[:NBUF], sc[NBUF:2 * NBUF], sc[2 * NBUF:3 * NBUF]
        msg, agg_sh, isem, gsem, ssem = sc[3 * NBUF:]
        cid = lax.axis_index("c")
        sid = lax.axis_index("s")
        wid = cid * NS + sid
        base0 = wid * EDGES_PER_TILE

        def idx_copies(i, s):
            off = base0 + i * K_EDGE
            return (
                pltpu.make_async_copy(
                    gidx.at[pl.ds(off, K_EDGE)], gbufs[s], isem.at[s]),
                pltpu.make_async_copy(
                    dst.at[pl.ds(off, K_EDGE)], dbufs[s], isem.at[s]),
                pltpu.make_async_copy(
                    norm.at[pl.ds(off, K_EDGE)], nbufs[s], isem.at[s]),
            )

        def idx_load(i, s):
            for cp in idx_copies(i, s):
                cp.start()

        def idx_wait(s):
            for cp in idx_copies(0, s):
                cp.wait()

        def gather(s):
            return pltpu.make_async_copy(
                table.at[gbufs[s]], msg.at[s], gsem.at[s])

        def scatter(s):
            return pltpu.make_async_copy(
                msg.at[s], agg_sh.at[dbufs[s]], ssem.at[s])

        pltpu.sync_copy(zeros.at[pl.ds(sid * N_PER_TILE, N_PER_TILE)],
                        agg_sh.at[pl.ds(sid * N_PER_TILE, N_PER_TILE)])
        plsc.subcore_barrier()

        idx_load(0, 0)
        idx_load(1, 1)
        idx_load(2, 2)
        idx_wait(0)
        gather(0).start()
        idx_wait(1)
        gather(1).start()

        def group(g, carry):
            for b in range(NBUF):
                i = g * NBUF + b
                s_m1 = (b + NBUF - 1) % NBUF
                s_p2 = (b + 2) % NBUF
                gather(b).wait()

                def scale(q, c2):
                    nv = nbufs[b][pl.ds(q * LANES, LANES)]
                    for t in range(LANES):
                        nj = nv[t]
                        j = q * LANES + t
                        for w in range(d // LANES):
                            sl = pl.ds(w * LANES, LANES)
                            msg[b, j, sl] = msg[b, j, sl] * nj
                    return c2

                lax.fori_loop(0, K_EDGE // LANES, scale, 0)
                scatter(b).start(add=True)

                @pl.when(i > 0)
                def _():
                    scatter(s_m1).wait()

                @pl.when(i + 3 < NCH)
                def _():
                    idx_load(i + 3, s_m1)

                @pl.when(i + 2 < NCH)
                def _():
                    idx_wait(s_p2)
                    gather(s_p2).start()

            return carry

        lax.fori_loop(0, NCH // NBUF, group, 0)
        scatter(NBUF - 1).wait()
        plsc.subcore_barrier()
        pltpu.sync_copy(agg_sh.at[pl.ds(sid * N_PER_TILE, N_PER_TILE)],
                        out.at[cid, pl.ds(sid * N_PER_TILE, N_PER_TILE)])

    return sc_scatter


def kernel(x, edge_index, edge_types, norm, bases1, comp1, loop_w1, bias1,
           bases2, comp2, loop_w2, bias2):
    src = edge_index[0]
    dst = edge_index[1]
    ar = jnp.arange(N, dtype=jnp.int32)
    pad = E_PAD - E_TOT

    arp = jnp.arange(pad, dtype=jnp.int32)
    dst_all = jnp.concatenate([dst, ar, arp % N_PAD])
    norm_all = jnp.concatenate(
        [norm, jnp.ones((N,), jnp.float32), jnp.zeros((pad,), jnp.float32)])
    gidx = jnp.concatenate([edge_types * N + src, R * N + ar, arp % N])

    comp_ext1 = jnp.concatenate([
        jnp.concatenate([comp1, jnp.zeros((R, 1), jnp.float32)], axis=1),
        jnp.concatenate([jnp.zeros((1, C), jnp.float32),
                         jnp.ones((1, 1), jnp.float32)], axis=1),
    ], axis=0)
    comp_ext2 = jnp.concatenate([
        jnp.concatenate([comp2, jnp.zeros((R, 1), jnp.float32)], axis=1),
        jnp.concatenate([jnp.zeros((1, C), jnp.float32),
                         jnp.ones((1, 1), jnp.float32)], axis=1),
    ], axis=0)
    bases1_ext = jnp.concatenate([bases1, loop_w1[None]], axis=0)
    bases2_ext = jnp.concatenate([bases2, loop_w2[None]], axis=0)

    zeros128 = jnp.zeros((N_PAD, HID), jnp.float32)
    scat = _make_sc_scatter(HID)

    h1_tab = _proj(comp_ext1, bases1_ext, x)
    p1 = scat(h1_tab.reshape((R + 1) * N, HID), gidx, dst_all, norm_all,
              zeros128)
    h1 = _combine(p1, bias1.reshape(1, HID), relu=True, bn=1000, nrows=N,
                  dout=HID)

    h2_tab = _proj(comp_ext2, bases2_ext, h1)
    p2 = scat(h2_tab.reshape((R + 1) * N, HID), gidx, dst_all, norm_all,
              zeros128)
    return _combine(p2, bias2.reshape(1, CLS), relu=False, bn=1000, nrows=N,
                    dout=CLS)


# speedup vs baseline: 2.5805x; 1.0505x over previous
import functools

import jax
import jax.numpy as jnp
from jax import lax
from jax.experimental import pallas as pl
from jax.experimental.pallas import tpu as pltpu
from jax.experimental.pallas import tpu_sc as plsc

N = 10000
E = 320000
IN = 128
HID = 128
CLS = 16
R = 8
C = 4

NC = 2
NS = 16
LANES = 16
NW = NC * NS

K_EDGE = 80
NBUF = 4
E_TOT = E + N
NCH = -(-E_TOT // (NW * K_EDGE * NBUF)) * NBUF
EDGES_PER_TILE = NCH * K_EDGE
E_PAD = EDGES_PER_TILE * NW
N_PAD = 10240
N_PER_TILE = N_PAD // NS


def _proj_body(ck, d, pad, comp_ref, bases_ref, h_ref, out_ref):
    r = pl.program_id(1)
    w = comp_ref[r, 0] * bases_ref[0]
    for c in range(1, ck):
        w = w + comp_ref[r, c] * bases_ref[c]
    m = jnp.dot(h_ref[...].astype(jnp.bfloat16), w.astype(jnp.bfloat16),
                preferred_element_type=jnp.float32)
    if pad:
        m = jnp.concatenate(
            [m, jnp.zeros((m.shape[0], pad), jnp.float32)], axis=1)
    out_ref[0] = m


def _proj(comp_ext, bases_ext, h, bn=1000):
    rk, ck = comp_ext.shape
    d = bases_ext.shape[-1]
    nb = N // bn
    return pl.pallas_call(
        functools.partial(_proj_body, ck, d, HID - d),
        grid=(nb, rk),
        in_specs=[
            pl.BlockSpec(memory_space=pltpu.SMEM),
            pl.BlockSpec((ck, IN, d), lambda b, r: (0, 0, 0)),
            pl.BlockSpec((bn, IN), lambda b, r: (b, 0)),
        ],
        out_specs=pl.BlockSpec((1, bn, HID), lambda b, r: (r, b, 0)),
        out_shape=jax.ShapeDtypeStruct((rk, N, HID), jnp.float32),
    )(comp_ext, bases_ext, h)


def _combine_body(relu, dout, p_ref, b_ref, out_ref):
    acc = p_ref[0] + p_ref[1]
    acc = acc[:, :dout] + b_ref[...]
    out_ref[...] = jnp.maximum(acc, 0.0) if relu else acc


def _combine(partials, bias_row, relu, bn, nrows, dout):
    nb = nrows // bn
    d = partials.shape[-1]
    return pl.pallas_call(
        functools.partial(_combine_body, relu, dout),
        grid=(nb,),
        in_specs=[
            pl.BlockSpec((2, bn, d), lambda b: (0, b, 0)),
            pl.BlockSpec((1, dout), lambda b: (0, 0)),
        ],
        out_specs=pl.BlockSpec((bn, dout), lambda b: (b, 0)),
        out_shape=jax.ShapeDtypeStruct((nrows, dout), jnp.float32),
    )(partials, bias_row)


@functools.lru_cache(maxsize=None)
def _make_sc_scatter(d):
    mesh = plsc.VectorSubcoreMesh(
        core_axis_name="c", subcore_axis_name="s", num_cores=NC,
        num_subcores=NS)

    @functools.partial(
        pl.kernel,
        out_type=jax.ShapeDtypeStruct((NC, N_PAD, d), jnp.float32),
        mesh=mesh,
        scratch_types=(
            [pltpu.VMEM((K_EDGE,), jnp.int32) for _ in range(3 * NBUF)]
            + [pltpu.VMEM((K_EDGE,), jnp.float32) for _ in range(NBUF)]
            + [
                pltpu.VMEM((NBUF, K_EDGE, d), jnp.float32),
                pltpu.VMEM_SHARED((N_PAD, d), jnp.float32),
                pltpu.SemaphoreType.DMA((NBUF,)),
                pltpu.SemaphoreType.DMA((NBUF,)),
                pltpu.SemaphoreType.DMA((NBUF,)),
            ]
        ),
    )
    def sc_scatter(table, src_e, dst_e, et_e, norm_e, zeros, out, *sc):
        gbufs, dbufs = sc[:NBUF], sc[NBUF:2 * NBUF]
        ebufs, nbufs = sc[2 * NBUF:3 * NBUF], sc[3 * NBUF:4 * NBUF]
        msg, agg_sh, isem, gsem, ssem = sc[4 * NBUF:]
        cid = lax.axis_index("c")
        sid = lax.axis_index("s")
        wid = cid * NS + sid
        base0 = wid * EDGES_PER_TILE

        def idx_copies(i, s):
            off = base0 + i * K_EDGE
            return (
                pltpu.make_async_copy(
                    src_e.at[pl.ds(off, K_EDGE)], gbufs[s], isem.at[s]),
                pltpu.make_async_copy(
                    dst_e.at[pl.ds(off, K_EDGE)], dbufs[s], isem.at[s]),
                pltpu.make_async_copy(
                    et_e.at[pl.ds(off, K_EDGE)], ebufs[s], isem.at[s]),
                pltpu.make_async_copy(
                    norm_e.at[pl.ds(off, K_EDGE)], nbufs[s], isem.at[s]),
            )

        def idx_load(i, s):
            off = base0 + i * K_EDGE

            @pl.when(off < E)
            def _():
                for cp in idx_copies(i, s):
                    cp.start()

        def idx_wait(i, s):
            off = base0 + i * K_EDGE

            @pl.when(off < E)
            def _():
                for cp in idx_copies(i, s):
                    cp.wait()
                for q in range(K_EDGE // LANES):
                    sl = pl.ds(q * LANES, LANES)
                    gbufs[s][sl] = ebufs[s][sl] * N + gbufs[s][sl]

            @pl.when(off >= E)
            def _():
                is_self = off < E + N
                base = jnp.where(is_self, off - E, off - (E + N))
                gbase = jnp.where(is_self, R * N + base, base)
                nval = jnp.where(is_self, 1.0, 0.0).astype(jnp.float32)
                for q in range(K_EDGE // LANES):
                    sl = pl.ds(q * LANES, LANES)
                    j = q * LANES + lax.iota(jnp.int32, LANES)
                    gbufs[s][sl] = gbase + j
                    dbufs[s][sl] = base + j
                    nbufs[s][sl] = jnp.zeros((LANES,), jnp.float32) + nval

        def gather(s):
            return pltpu.make_async_copy(
                table.at[gbufs[s]], msg.at[s], gsem.at[s])

        def scatter(s):
            return pltpu.make_async_copy(
                msg.at[s], agg_sh.at[dbufs[s]], ssem.at[s])

        pltpu.sync_copy(zeros.at[pl.ds(sid * N_PER_TILE, N_PER_TILE)],
                        agg_sh.at[pl.ds(sid * N_PER_TILE, N_PER_TILE)])
        plsc.subcore_barrier()

        idx_load(0, 0)
        idx_load(1, 1)
        idx_load(2, 2)
        idx_wait(0, 0)
        gather(0).start()
        idx_wait(1, 1)
        gather(1).start()

        def group(g, carry):
            for b in range(NBUF):
                i = g * NBUF + b
                s_m1 = (b + NBUF - 1) % NBUF
                s_p2 = (b + 2) % NBUF
                gather(b).wait()

                def scale(q, c2):
                    nv = nbufs[b][pl.ds(q * LANES, LANES)]
                    for t in range(LANES):
                        nj = nv[t]
                        j = q * LANES + t
                        for w in range(d // LANES):
                            sl = pl.ds(w * LANES, LANES)
                            msg[b, j, sl] = msg[b, j, sl] * nj
                    return c2

                lax.fori_loop(0, K_EDGE // LANES, scale, 0)
                scatter(b).start(add=True)

                @pl.when(i > 0)
                def _():
                    scatter(s_m1).wait()

                @pl.when(i + 3 < NCH)
                def _():
                    idx_load(i + 3, s_m1)

                @pl.when(i + 2 < NCH)
                def _():
                    idx_wait(i + 2, s_p2)
                    gather(s_p2).start()

            return carry

        lax.fori_loop(0, NCH // NBUF, group, 0)
        scatter(NBUF - 1).wait()
        plsc.subcore_barrier()
        pltpu.sync_copy(agg_sh.at[pl.ds(sid * N_PER_TILE, N_PER_TILE)],
                        out.at[cid, pl.ds(sid * N_PER_TILE, N_PER_TILE)])

    return sc_scatter


def kernel(x, edge_index, edge_types, norm, bases1, comp1, loop_w1, bias1,
           bases2, comp2, loop_w2, bias2):
    src = edge_index[0]
    dst = edge_index[1]
    comp_ext1 = jnp.concatenate([
        jnp.concatenate([comp1, jnp.zeros((R, 1), jnp.float32)], axis=1),
        jnp.concatenate([jnp.zeros((1, C), jnp.float32),
                         jnp.ones((1, 1), jnp.float32)], axis=1),
    ], axis=0)
    comp_ext2 = jnp.concatenate([
        jnp.concatenate([comp2, jnp.zeros((R, 1), jnp.float32)], axis=1),
        jnp.concatenate([jnp.zeros((1, C), jnp.float32),
                         jnp.ones((1, 1), jnp.float32)], axis=1),
    ], axis=0)
    bases1_ext = jnp.concatenate([bases1, loop_w1[None]], axis=0)
    bases2_ext = jnp.concatenate([bases2, loop_w2[None]], axis=0)

    zeros128 = jnp.zeros((N_PAD, HID), jnp.float32)
    scat = _make_sc_scatter(HID)

    h1_tab = _proj(comp_ext1, bases1_ext, x)
    p1 = scat(h1_tab.reshape((R + 1) * N, HID), src, dst, edge_types, norm,
              zeros128)
    h1 = _combine(p1, bias1.reshape(1, HID), relu=True, bn=1000, nrows=N,
                  dout=HID)

    h2_tab = _proj(comp_ext2, bases2_ext, h1)
    p2 = scat(h2_tab.reshape((R + 1) * N, HID), src, dst, edge_types, norm,
              zeros128)
    return _combine(p2, bias2.reshape(1, CLS), relu=False, bn=1000, nrows=N,
                    dout=CLS)


# speedup vs baseline: 2.7074x; 1.0492x over previous
import functools

import jax
import jax.numpy as jnp
from jax import lax
from jax.experimental import pallas as pl
from jax.experimental.pallas import tpu as pltpu
from jax.experimental.pallas import tpu_sc as plsc

N = 10000
E = 320000
IN = 128
HID = 128
CLS = 16
R = 8
C = 4

NC = 2
NS = 16
LANES = 16
NW = NC * NS

K_EDGE = 80
NBUF = 4
E_TOT = E + N
NCH = -(-E_TOT // (NW * K_EDGE * NBUF)) * NBUF
EDGES_PER_TILE = NCH * K_EDGE
E_PAD = EDGES_PER_TILE * NW
N_PAD = 10240
N_PER_TILE = N_PAD // NS


def _proj_body(ck, d, pad, comp_ref, bases_ref, h_ref, out_ref):
    r = pl.program_id(1)
    w = comp_ref[r, 0] * bases_ref[0]
    for c in range(1, ck):
        w = w + comp_ref[r, c] * bases_ref[c]
    m = jnp.dot(h_ref[...].astype(jnp.bfloat16), w.astype(jnp.bfloat16),
                preferred_element_type=jnp.float32)
    if pad:
        m = jnp.concatenate(
            [m, jnp.zeros((m.shape[0], pad), jnp.float32)], axis=1)
    out_ref[0] = m


def _proj(comp_ext, bases_ext, h, bn=1000):
    rk, ck = comp_ext.shape
    d = bases_ext.shape[-1]
    nb = N // bn
    return pl.pallas_call(
        functools.partial(_proj_body, ck, d, HID - d),
        grid=(nb, rk),
        in_specs=[
            pl.BlockSpec(memory_space=pltpu.SMEM),
            pl.BlockSpec((ck, IN, d), lambda b, r: (0, 0, 0)),
            pl.BlockSpec((bn, IN), lambda b, r: (b, 0)),
        ],
        out_specs=pl.BlockSpec((1, bn, HID), lambda b, r: (r, b, 0)),
        out_shape=jax.ShapeDtypeStruct((rk, N, HID), jnp.float32),
    )(comp_ext, bases_ext, h)


def _combine_body(relu, dout, p_ref, b_ref, out_ref):
    acc = p_ref[0] + p_ref[1]
    acc = acc[:, :dout] + b_ref[...]
    out_ref[...] = jnp.maximum(acc, 0.0) if relu else acc


def _combine(partials, bias_row, relu, bn, nrows, dout):
    nb = nrows // bn
    d = partials.shape[-1]
    return pl.pallas_call(
        functools.partial(_combine_body, relu, dout),
        grid=(nb,),
        in_specs=[
            pl.BlockSpec((2, bn, d), lambda b: (0, b, 0)),
            pl.BlockSpec((1, dout), lambda b: (0, 0)),
        ],
        out_specs=pl.BlockSpec((bn, dout), lambda b: (b, 0)),
        out_shape=jax.ShapeDtypeStruct((nrows, dout), jnp.float32),
    )(partials, bias_row)


@functools.lru_cache(maxsize=None)
def _make_sc_scatter(d, scale_w):
    mesh = plsc.VectorSubcoreMesh(
        core_axis_name="c", subcore_axis_name="s", num_cores=NC,
        num_subcores=NS)

    @functools.partial(
        pl.kernel,
        out_type=jax.ShapeDtypeStruct((NC, N_PAD, d), jnp.float32),
        mesh=mesh,
        scratch_types=(
            [pltpu.VMEM((K_EDGE,), jnp.int32) for _ in range(3 * NBUF)]
            + [pltpu.VMEM((K_EDGE,), jnp.float32) for _ in range(NBUF)]
            + [
                pltpu.VMEM((NBUF, K_EDGE, d), jnp.float32),
                pltpu.VMEM_SHARED((N_PAD, d), jnp.float32),
                pltpu.SemaphoreType.DMA((NBUF,)),
                pltpu.SemaphoreType.DMA((NBUF,)),
                pltpu.SemaphoreType.DMA((NBUF,)),
            ]
        ),
    )
    def sc_scatter(table, src_e, dst_e, et_e, norm_e, zeros, out, *sc):
        gbufs, dbufs = sc[:NBUF], sc[NBUF:2 * NBUF]
        ebufs, nbufs = sc[2 * NBUF:3 * NBUF], sc[3 * NBUF:4 * NBUF]
        msg, agg_sh, isem, gsem, ssem = sc[4 * NBUF:]
        cid = lax.axis_index("c")
        sid = lax.axis_index("s")
        wid = cid * NS + sid
        base0 = wid * EDGES_PER_TILE

        def idx_copies(i, s):
            off = base0 + i * K_EDGE
            return (
                pltpu.make_async_copy(
                    src_e.at[pl.ds(off, K_EDGE)], gbufs[s], isem.at[s]),
                pltpu.make_async_copy(
                    dst_e.at[pl.ds(off, K_EDGE)], dbufs[s], isem.at[s]),
                pltpu.make_async_copy(
                    et_e.at[pl.ds(off, K_EDGE)], ebufs[s], isem.at[s]),
                pltpu.make_async_copy(
                    norm_e.at[pl.ds(off, K_EDGE)], nbufs[s], isem.at[s]),
            )

        def idx_load(i, s):
            off = base0 + i * K_EDGE

            @pl.when(off < E)
            def _():
                for cp in idx_copies(i, s):
                    cp.start()

        def idx_wait(i, s):
            off = base0 + i * K_EDGE

            @pl.when(off < E)
            def _():
                for cp in idx_copies(i, s):
                    cp.wait()
                for q in range(K_EDGE // LANES):
                    sl = pl.ds(q * LANES, LANES)
                    gbufs[s][sl] = ebufs[s][sl] * N + gbufs[s][sl]

            @pl.when(off >= E)
            def _():
                is_self = off < E + N
                base = jnp.where(is_self, off - E, off - (E + N))
                gbase = jnp.where(is_self, R * N + base, base)
                nval = jnp.where(is_self, 1.0, 0.0).astype(jnp.float32)
                for q in range(K_EDGE // LANES):
                    sl = pl.ds(q * LANES, LANES)
                    j = q * LANES + lax.iota(jnp.int32, LANES)
                    gbufs[s][sl] = gbase + j
                    dbufs[s][sl] = base + j
                    nbufs[s][sl] = jnp.zeros((LANES,), jnp.float32) + nval

        def gather(s):
            return pltpu.make_async_copy(
                table.at[gbufs[s]], msg.at[s], gsem.at[s])

        def scatter(s):
            return pltpu.make_async_copy(
                msg.at[s], agg_sh.at[dbufs[s]], ssem.at[s])

        pltpu.sync_copy(zeros.at[pl.ds(sid * N_PER_TILE, N_PER_TILE)],
                        agg_sh.at[pl.ds(sid * N_PER_TILE, N_PER_TILE)])
        plsc.subcore_barrier()

        idx_load(0, 0)
        idx_load(1, 1)
        idx_load(2, 2)
        idx_wait(0, 0)
        gather(0).start()
        idx_wait(1, 1)
        gather(1).start()

        def group(g, carry):
            for b in range(NBUF):
                i = g * NBUF + b
                s_m1 = (b + NBUF - 1) % NBUF
                s_p2 = (b + 2) % NBUF
                gather(b).wait()

                def scale(q, c2):
                    nv = nbufs[b][pl.ds(q * LANES, LANES)]
                    for t in range(LANES):
                        nj = nv[t]
                        j = q * LANES + t
                        for w in range(scale_w // LANES):
                            sl = pl.ds(w * LANES, LANES)
                            msg[b, j, sl] = msg[b, j, sl] * nj
                    return c2

                lax.fori_loop(0, K_EDGE // LANES, scale, 0)
                scatter(b).start(add=True)

                @pl.when(i > 0)
                def _():
                    scatter(s_m1).wait()

                @pl.when(i + 3 < NCH)
                def _():
                    idx_load(i + 3, s_m1)

                @pl.when(i + 2 < NCH)
                def _():
                    idx_wait(i + 2, s_p2)
                    gather(s_p2).start()

            return carry

        lax.fori_loop(0, NCH // NBUF, group, 0)
        scatter(NBUF - 1).wait()
        plsc.subcore_barrier()
        pltpu.sync_copy(agg_sh.at[pl.ds(sid * N_PER_TILE, N_PER_TILE)],
                        out.at[cid, pl.ds(sid * N_PER_TILE, N_PER_TILE)])

    return sc_scatter


def kernel(x, edge_index, edge_types, norm, bases1, comp1, loop_w1, bias1,
           bases2, comp2, loop_w2, bias2):
    src = edge_index[0]
    dst = edge_index[1]
    comp_ext1 = jnp.concatenate([
        jnp.concatenate([comp1, jnp.zeros((R, 1), jnp.float32)], axis=1),
        jnp.concatenate([jnp.zeros((1, C), jnp.float32),
                         jnp.ones((1, 1), jnp.float32)], axis=1),
    ], axis=0)
    comp_ext2 = jnp.concatenate([
        jnp.concatenate([comp2, jnp.zeros((R, 1), jnp.float32)], axis=1),
        jnp.concatenate([jnp.zeros((1, C), jnp.float32),
                         jnp.ones((1, 1), jnp.float32)], axis=1),
    ], axis=0)
    bases1_ext = jnp.concatenate([bases1, loop_w1[None]], axis=0)
    bases2_ext = jnp.concatenate([bases2, loop_w2[None]], axis=0)

    zeros128 = jnp.zeros((N_PAD, HID), jnp.float32)

    h1_tab = _proj(comp_ext1, bases1_ext, x)
    p1 = _make_sc_scatter(HID, HID)(
        h1_tab.reshape((R + 1) * N, HID), src, dst, edge_types, norm,
        zeros128)
    h1 = _combine(p1, bias1.reshape(1, HID), relu=True, bn=1000, nrows=N,
                  dout=HID)

    h2_tab = _proj(comp_ext2, bases2_ext, h1)
    p2 = _make_sc_scatter(HID, CLS)(
        h2_tab.reshape((R + 1) * N, HID), src, dst, edge_types, norm,
        zeros128)
    return _combine(p2, bias2.reshape(1, CLS), relu=False, bn=1000, nrows=N,
                    dout=CLS)


# speedup vs baseline: 2.7528x; 1.0168x over previous
import functools

import jax
import jax.numpy as jnp
from jax import lax
from jax.experimental import pallas as pl
from jax.experimental.pallas import tpu as pltpu
from jax.experimental.pallas import tpu_sc as plsc

N = 10000
E = 320000
IN = 128
HID = 128
CLS = 16
R = 8
C = 4

NC = 2
NS = 16
LANES = 16
NW = NC * NS

K_EDGE = 80
NBUF = 4
E_TOT = E + N
NCH = -(-E_TOT // (NW * K_EDGE * NBUF)) * NBUF
EDGES_PER_TILE = NCH * K_EDGE
E_PAD = EDGES_PER_TILE * NW
N_PAD = 10240
N_PER_TILE = N_PAD // NS


def _proj_body(ck, d, pad, comp_ref, bases_ref, h_ref, out_ref):
    r = pl.program_id(1)
    w = comp_ref[r, 0] * bases_ref[0]
    for c in range(1, ck):
        w = w + comp_ref[r, c] * bases_ref[c]
    m = jnp.dot(h_ref[...].astype(jnp.bfloat16), w.astype(jnp.bfloat16),
                preferred_element_type=jnp.float32)
    if pad:
        m = jnp.concatenate(
            [m, jnp.zeros((m.shape[0], pad), jnp.float32)], axis=1)
    out_ref[0] = m


def _proj(comp_ext, bases_ext, h, bn=1000):
    rk, ck = comp_ext.shape
    d = bases_ext.shape[-1]
    nb = N // bn
    return pl.pallas_call(
        functools.partial(_proj_body, ck, d, HID - d),
        grid=(nb, rk),
        in_specs=[
            pl.BlockSpec(memory_space=pltpu.SMEM),
            pl.BlockSpec((ck, IN, d), lambda b, r: (0, 0, 0)),
            pl.BlockSpec((bn, IN), lambda b, r: (b, 0)),
        ],
        out_specs=pl.BlockSpec((1, bn, HID), lambda b, r: (r, b, 0)),
        out_shape=jax.ShapeDtypeStruct((rk, N, HID), jnp.float32),
    )(comp_ext, bases_ext, h)


def _combine_body(relu, dout, p_ref, b_ref, out_ref):
    acc = p_ref[0] + p_ref[1]
    acc = acc[:, :dout] + b_ref[...]
    out_ref[...] = jnp.maximum(acc, 0.0) if relu else acc


def _combine(partials, bias_row, relu, bn, nrows, dout):
    nb = nrows // bn
    d = partials.shape[-1]
    return pl.pallas_call(
        functools.partial(_combine_body, relu, dout),
        grid=(nb,),
        in_specs=[
            pl.BlockSpec((2, bn, d), lambda b: (0, b, 0)),
            pl.BlockSpec((1, dout), lambda b: (0, 0)),
        ],
        out_specs=pl.BlockSpec((bn, dout), lambda b: (b, 0)),
        out_shape=jax.ShapeDtypeStruct((nrows, dout), jnp.float32),
    )(partials, bias_row)


def _proj2_body(ck, d, pad, comp_ref, bases_ref, p_ref, b_ref, out_ref, h1s):
    r = pl.program_id(1)

    @pl.when(r == 0)
    def _():
        h1s[...] = jnp.maximum(p_ref[0] + p_ref[1] + b_ref[...], 0.0)

    w = comp_ref[r, 0] * bases_ref[0]
    for c in range(1, ck):
        w = w + comp_ref[r, c] * bases_ref[c]
    m = jnp.dot(h1s[...].astype(jnp.bfloat16), w.astype(jnp.bfloat16),
                preferred_element_type=jnp.float32)
    if pad:
        m = jnp.concatenate(
            [m, jnp.zeros((m.shape[0], pad), jnp.float32)], axis=1)
    out_ref[0] = m


def _proj2(comp_ext, bases_ext, p1, bias_row, bn=1000):
    rk, ck = comp_ext.shape
    d = bases_ext.shape[-1]
    nb = N // bn
    return pl.pallas_call(
        functools.partial(_proj2_body, ck, d, HID - d),
        grid=(nb, rk),
        in_specs=[
            pl.BlockSpec(memory_space=pltpu.SMEM),
            pl.BlockSpec((ck, IN, d), lambda b, r: (0, 0, 0)),
            pl.BlockSpec((2, bn, IN), lambda b, r: (0, b, 0)),
            pl.BlockSpec((1, IN), lambda b, r: (0, 0)),
        ],
        out_specs=pl.BlockSpec((1, bn, HID), lambda b, r: (r, b, 0)),
        out_shape=jax.ShapeDtypeStruct((rk, N, HID), jnp.float32),
        scratch_shapes=[pltpu.VMEM((bn, IN), jnp.float32)],
    )(comp_ext, bases_ext, p1, bias_row)


@functools.lru_cache(maxsize=None)
def _make_sc_scatter(d, scale_w):
    mesh = plsc.VectorSubcoreMesh(
        core_axis_name="c", subcore_axis_name="s", num_cores=NC,
        num_subcores=NS)

    @functools.partial(
        pl.kernel,
        out_type=jax.ShapeDtypeStruct((NC, N_PAD, d), jnp.float32),
        mesh=mesh,
        scratch_types=(
            [pltpu.VMEM((K_EDGE,), jnp.int32) for _ in range(3 * NBUF)]
            + [pltpu.VMEM((K_EDGE,), jnp.float32) for _ in range(NBUF)]
            + [
                pltpu.VMEM((NBUF, K_EDGE, d), jnp.float32),
                pltpu.VMEM_SHARED((N_PAD, d), jnp.float32),
                pltpu.SemaphoreType.DMA((NBUF,)),
                pltpu.SemaphoreType.DMA((NBUF,)),
                pltpu.SemaphoreType.DMA((NBUF,)),
            ]
        ),
    )
    def sc_scatter(table, src_e, dst_e, et_e, norm_e, zeros, out, *sc):
        gbufs, dbufs = sc[:NBUF], sc[NBUF:2 * NBUF]
        ebufs, nbufs = sc[2 * NBUF:3 * NBUF], sc[3 * NBUF:4 * NBUF]
        msg, agg_sh, isem, gsem, ssem = sc[4 * NBUF:]
        cid = lax.axis_index("c")
        sid = lax.axis_index("s")
        wid = cid * NS + sid
        base0 = wid * EDGES_PER_TILE

        def idx_copies(i, s):
            off = base0 + i * K_EDGE
            return (
                pltpu.make_async_copy(
                    src_e.at[pl.ds(off, K_EDGE)], gbufs[s], isem.at[s]),
                pltpu.make_async_copy(
                    dst_e.at[pl.ds(off, K_EDGE)], dbufs[s], isem.at[s]),
                pltpu.make_async_copy(
                    et_e.at[pl.ds(off, K_EDGE)], ebufs[s], isem.at[s]),
                pltpu.make_async_copy(
                    norm_e.at[pl.ds(off, K_EDGE)], nbufs[s], isem.at[s]),
            )

        def idx_load(i, s):
            off = base0 + i * K_EDGE

            @pl.when(off < E)
            def _():
                for cp in idx_copies(i, s):
                    cp.start()

        def idx_wait(i, s):
            off = base0 + i * K_EDGE

            @pl.when(off < E)
            def _():
                for cp in idx_copies(i, s):
                    cp.wait()
                for q in range(K_EDGE // LANES):
                    sl = pl.ds(q * LANES, LANES)
                    gbufs[s][sl] = ebufs[s][sl] * N + gbufs[s][sl]

            @pl.when(off >= E)
            def _():
                is_self = off < E + N
                base = jnp.where(is_self, off - E, off - (E + N))
                gbase = jnp.where(is_self, R * N + base, base)
                nval = jnp.where(is_self, 1.0, 0.0).astype(jnp.float32)
                for q in range(K_EDGE // LANES):
                    sl = pl.ds(q * LANES, LANES)
                    j = q * LANES + lax.iota(jnp.int32, LANES)
                    gbufs[s][sl] = gbase + j
                    dbufs[s][sl] = base + j
                    nbufs[s][sl] = jnp.zeros((LANES,), jnp.float32) + nval

        def gather(s):
            return pltpu.make_async_copy(
                table.at[gbufs[s]], msg.at[s], gsem.at[s])

        def scatter(s):
            return pltpu.make_async_copy(
                msg.at[s], agg_sh.at[dbufs[s]], ssem.at[s])

        pltpu.sync_copy(zeros.at[pl.ds(sid * N_PER_TILE, N_PER_TILE)],
                        agg_sh.at[pl.ds(sid * N_PER_TILE, N_PER_TILE)])
        plsc.subcore_barrier()

        idx_load(0, 0)
        idx_load(1, 1)
        idx_load(2, 2)
        idx_wait(0, 0)
        gather(0).start()
        idx_wait(1, 1)
        gather(1).start()

        def group(g, carry):
            for b in range(NBUF):
                i = g * NBUF + b
                s_m1 = (b + NBUF - 1) % NBUF
                s_p2 = (b + 2) % NBUF
                gather(b).wait()

                def scale(q, c2):
                    nv = nbufs[b][pl.ds(q * LANES, LANES)]
                    for t in range(LANES):
                        nj = nv[t]
                        j = q * LANES + t
                        for w in range(scale_w // LANES):
                            sl = pl.ds(w * LANES, LANES)
                            msg[b, j, sl] = msg[b, j, sl] * nj
                    return c2

                lax.fori_loop(0, K_EDGE // LANES, scale, 0)
                scatter(b).start(add=True)

                @pl.when(i > 0)
                def _():
                    scatter(s_m1).wait()

                @pl.when(i + 3 < NCH)
                def _():
                    idx_load(i + 3, s_m1)

                @pl.when(i + 2 < NCH)
                def _():
                    idx_wait(i + 2, s_p2)
                    gather(s_p2).start()

            return carry

        lax.fori_loop(0, NCH // NBUF, group, 0)
        scatter(NBUF - 1).wait()
        plsc.subcore_barrier()
        pltpu.sync_copy(agg_sh.at[pl.ds(sid * N_PER_TILE, N_PER_TILE)],
                        out.at[cid, pl.ds(sid * N_PER_TILE, N_PER_TILE)])

    return sc_scatter


def kernel(x, edge_index, edge_types, norm, bases1, comp1, loop_w1, bias1,
           bases2, comp2, loop_w2, bias2):
    src = edge_index[0]
    dst = edge_index[1]
    comp_ext1 = jnp.concatenate([
        jnp.concatenate([comp1, jnp.zeros((R, 1), jnp.float32)], axis=1),
        jnp.concatenate([jnp.zeros((1, C), jnp.float32),
                         jnp.ones((1, 1), jnp.float32)], axis=1),
    ], axis=0)
    comp_ext2 = jnp.concatenate([
        jnp.concatenate([comp2, jnp.zeros((R, 1), jnp.float32)], axis=1),
        jnp.concatenate([jnp.zeros((1, C), jnp.float32),
                         jnp.ones((1, 1), jnp.float32)], axis=1),
    ], axis=0)
    bases1_ext = jnp.concatenate([bases1, loop_w1[None]], axis=0)
    bases2_ext = jnp.concatenate([bases2, loop_w2[None]], axis=0)

    zeros128 = jnp.zeros((N_PAD, HID), jnp.float32)

    h1_tab = _proj(comp_ext1, bases1_ext, x)
    p1 = _make_sc_scatter(HID, HID)(
        h1_tab.reshape((R + 1) * N, HID), src, dst, edge_types, norm,
        zeros128)

    h2_tab = _proj2(comp_ext2, bases2_ext, p1, bias1.reshape(1, HID))
    p2 = _make_sc_scatter(HID, CLS)(
        h2_tab.reshape((R + 1) * N, HID), src, dst, edge_types, norm,
        zeros128)
    return _combine(p2, bias2.reshape(1, CLS), relu=False, bn=1000, nrows=N,
                    dout=CLS)
